# baseline (speedup 1.0000x reference)
import functools

import jax
import jax.numpy as jnp
from jax import lax
from jax.experimental import pallas as pl
from jax.experimental.pallas import tpu as pltpu
from jax.experimental.pallas import tpu_sc as plsc

_NC = 2
_NS = 16
_NW = _NC * _NS

_CHUNK = 262144


_NBUF = 3
_RA = 2


def _sc_body(n_chunks, in_hbm, out_hbm, wsem):
    wid = lax.axis_index("s") * _NC + lax.axis_index("c")
    base = wid * (n_chunks * _CHUNK)

    for g in range(n_chunks):
        pltpu.async_copy(
            in_hbm.at[pl.ds(base + g * _CHUNK, _CHUNK)],
            out_hbm.at[pl.ds(base + g * _CHUNK, _CHUNK)], wsem)
    for g in range(n_chunks):
        pltpu.make_async_copy(
            in_hbm.at[pl.ds(base + g * _CHUNK, _CHUNK)],
            out_hbm.at[pl.ds(base + g * _CHUNK, _CHUNK)], wsem
        ).wait()


def kernel(phi):
    shape = phi.shape
    flat = phi.reshape(-1)
    n = flat.shape[0]
    assert n % (_NW * _CHUNK) == 0
    n_chunks = n // (_NW * _CHUNK)

    mesh = plsc.VectorSubcoreMesh(core_axis_name="c", subcore_axis_name="s")
    run = pl.kernel(
        functools.partial(_sc_body, n_chunks),
        mesh=mesh,
        out_type=jax.ShapeDtypeStruct((n,), flat.dtype),
        scratch_types=[
            pltpu.SemaphoreType.DMA,
        ],
    )
    return run(flat).reshape(shape)


# speedup vs baseline: 12.2465x; 12.2465x over previous
import functools

import jax
import jax.numpy as jnp
from jax import lax
from jax.experimental import pallas as pl
from jax.experimental.pallas import tpu as pltpu
from jax.experimental.pallas import tpu_sc as plsc

_NC = 2
_NS = 16
_NW = _NC * _NS

_CHUNK = 524288
_NSLOT = 3
_RA = 2


def _sc_body(n_chunks, in_hbm, out_hbm, bufs, rsems, wsems):
    c = lax.axis_index("c")
    s = lax.axis_index("s")
    base = c * (n_chunks * _CHUNK)

    def read(g, slot):
        pltpu.async_copy(
            in_hbm.at[pl.ds(base + g * _CHUNK, _CHUNK)], bufs[slot],
            rsems[slot])

    def wait_read(g, slot):
        pltpu.make_async_copy(
            in_hbm.at[pl.ds(base + g * _CHUNK, _CHUNK)], bufs[slot],
            rsems[slot]).wait()

    def write(g, slot):
        pltpu.async_copy(
            bufs[slot], out_hbm.at[pl.ds(base + g * _CHUNK, _CHUNK)],
            wsems[slot])

    def wait_write(g, slot):
        pltpu.make_async_copy(
            bufs[slot], out_hbm.at[pl.ds(base + g * _CHUNK, _CHUNK)],
            wsems[slot]).wait()

    @pl.when(s == 0)
    def _():
        ra = min(_RA, n_chunks)
        for g in range(ra):
            read(g, g % _NSLOT)
        for g in range(n_chunks):
            slot = g % _NSLOT
            wait_read(g, slot)
            write(g, slot)
            nxt = g + ra
            if nxt < n_chunks:
                conflict = nxt - _NSLOT
                if conflict >= 0:
                    wait_write(conflict, conflict % _NSLOT)
                read(nxt, nxt % _NSLOT)
        for g in range(max(0, n_chunks - _NSLOT), n_chunks):
            wait_write(g, g % _NSLOT)


def kernel(phi):
    shape = phi.shape
    flat = phi.reshape(-1)
    n = flat.shape[0]
    assert n % (_NC * _CHUNK) == 0
    n_chunks = n // (_NC * _CHUNK)

    mesh = plsc.VectorSubcoreMesh(core_axis_name="c", subcore_axis_name="s")
    run = pl.kernel(
        functools.partial(_sc_body, n_chunks),
        mesh=mesh,
        out_type=jax.ShapeDtypeStruct((n,), flat.dtype),
        scratch_types=[
            [pltpu.VMEM_SHARED((_CHUNK,), jnp.float32)] * _NSLOT,
            [pltpu.SemaphoreType.DMA] * _NSLOT,
            [pltpu.SemaphoreType.DMA] * _NSLOT,
        ],
    )
    return run(flat).reshape(shape)


# speedup vs baseline: 38.7606x; 3.1650x over previous
import functools

import jax
import jax.numpy as jnp
from jax import lax
from jax.experimental import pallas as pl
from jax.experimental.pallas import tpu as pltpu
from jax.experimental.pallas import tpu_sc as plsc

_NC = 2
_NS = 16
_NW = _NC * _NS

_CIMG = 8
_NSLOT = 3
_RA = 2


def _sc_body(n_chunks, in_hbm, out_hbm, bufs, rsems, wsems):
    c = lax.axis_index("c")
    s = lax.axis_index("s")
    base = c * (n_chunks * _CIMG)

    def read(g, slot):
        pltpu.async_copy(
            in_hbm.at[pl.ds(base + g * _CIMG, _CIMG)], bufs[slot],
            rsems[slot])

    def wait_read(g, slot):
        pltpu.make_async_copy(
            in_hbm.at[pl.ds(base + g * _CIMG, _CIMG)], bufs[slot],
            rsems[slot]).wait()

    def write(g, slot):
        pltpu.async_copy(
            bufs[slot], out_hbm.at[pl.ds(base + g * _CIMG, _CIMG)],
            wsems[slot])

    def wait_write(g, slot):
        pltpu.make_async_copy(
            bufs[slot], out_hbm.at[pl.ds(base + g * _CIMG, _CIMG)],
            wsems[slot]).wait()

    @pl.when(s == 0)
    def _():
        ra = min(_RA, n_chunks)
        for g in range(ra):
            read(g, g % _NSLOT)
        for g in range(n_chunks):
            slot = g % _NSLOT
            wait_read(g, slot)
            write(g, slot)
            nxt = g + ra
            if nxt < n_chunks:
                conflict = nxt - _NSLOT
                if conflict >= 0:
                    wait_write(conflict, conflict % _NSLOT)
                read(nxt, nxt % _NSLOT)
        for g in range(max(0, n_chunks - _NSLOT), n_chunks):
            wait_write(g, g % _NSLOT)


def kernel(phi):
    B, H, W = phi.shape
    assert B % (_NC * _CIMG) == 0
    n_chunks = B // (_NC * _CIMG)

    mesh = plsc.VectorSubcoreMesh(core_axis_name="c", subcore_axis_name="s")
    run = pl.kernel(
        functools.partial(_sc_body, n_chunks),
        mesh=mesh,
        out_type=jax.ShapeDtypeStruct(phi.shape, phi.dtype),
        scratch_types=[
            [pltpu.VMEM_SHARED((_CIMG, H, W), jnp.float32)] * _NSLOT,
            [pltpu.SemaphoreType.DMA] * _NSLOT,
            [pltpu.SemaphoreType.DMA] * _NSLOT,
        ],
    )
    return run(phi)


# speedup vs baseline: 40.5910x; 1.0472x over previous
import functools

import jax
import jax.numpy as jnp
from jax import lax
from jax.experimental import pallas as pl
from jax.experimental.pallas import tpu as pltpu
from jax.experimental.pallas import tpu_sc as plsc

_NC = 2
_NS = 16
_NW = _NC * _NS

_CIMG = 4
_NSLOT = 3
_NDRV = 2
_RA = 2


def _sc_body(n_chunks, in_hbm, out_hbm, bufs, rsems, wsems):
    c = lax.axis_index("c")
    s = lax.axis_index("s")

    for d in range(_NDRV):
        @pl.when(s == d)
        def _(d=d):
            chunks = list(range(d, n_chunks, _NDRV))

            def img0(g):
                return (c * n_chunks + g) * _CIMG

            def read(g, slot):
                pltpu.async_copy(
                    in_hbm.at[pl.ds(img0(g), _CIMG)], bufs[slot], rsems[slot])

            def wait_read(g, slot):
                pltpu.make_async_copy(
                    in_hbm.at[pl.ds(img0(g), _CIMG)], bufs[slot],
                    rsems[slot]).wait()

            def write(g, slot):
                pltpu.async_copy(
                    bufs[slot], out_hbm.at[pl.ds(img0(g), _CIMG)], wsems[slot])

            def wait_write(g, slot):
                pltpu.make_async_copy(
                    bufs[slot], out_hbm.at[pl.ds(img0(g), _CIMG)],
                    wsems[slot]).wait()

            base_slot = d * _NSLOT
            n = len(chunks)
            ra = min(_RA, n)
            for k in range(ra):
                read(chunks[k], base_slot + k % _NSLOT)
            for k in range(n):
                slot = base_slot + k % _NSLOT
                wait_read(chunks[k], slot)
                write(chunks[k], slot)
                nk = k + ra
                if nk < n:
                    conflict = nk - _NSLOT
                    if conflict >= 0:
                        wait_write(chunks[conflict],
                                   base_slot + conflict % _NSLOT)
                    read(chunks[nk], base_slot + nk % _NSLOT)
            for k in range(max(0, n - _NSLOT), n):
                wait_write(chunks[k], base_slot + k % _NSLOT)


def kernel(phi):
    B, H, W = phi.shape
    assert B % (_NC * _CIMG) == 0
    n_chunks = B // (_NC * _CIMG)

    mesh = plsc.VectorSubcoreMesh(core_axis_name="c", subcore_axis_name="s")
    run = pl.kernel(
        functools.partial(_sc_body, n_chunks),
        mesh=mesh,
        out_type=jax.ShapeDtypeStruct(phi.shape, phi.dtype),
        scratch_types=[
            [pltpu.VMEM_SHARED((_CIMG, H, W), jnp.float32)] * (_NSLOT * _NDRV),
            [pltpu.SemaphoreType.DMA] * (_NSLOT * _NDRV),
            [pltpu.SemaphoreType.DMA] * (_NSLOT * _NDRV),
        ],
    )
    return run(phi)
